# baseline (speedup 1.0000x reference)
import functools

import jax
import jax.numpy as jnp
from jax import lax
from jax.experimental import pallas as pl
from jax.experimental.pallas import tpu as pltpu

_NBUF = 3


def _persistent_body(blocks, offs, w_ref, b_ref, x_hbm, adj_hbm, out_hbm,
                     xbuf, abuf, sup, obuf, x_sem, in_sems, out_sems):
    n = len(blocks)

    def adj_copy(j):
        off, sz = offs[j], blocks[j]
        return pltpu.make_async_copy(
            adj_hbm.at[pl.ds(off, sz)],
            abuf.at[j % _NBUF, pl.ds(0, sz)],
            in_sems.at[j % _NBUF])

    def out_copy(j):
        off, sz = offs[j], blocks[j]
        return pltpu.make_async_copy(
            obuf.at[j % 2, pl.ds(0, sz)],
            out_hbm.at[pl.ds(off, sz)],
            out_sems.at[j % 2])

    for j in range(min(_NBUF, n)):
        adj_copy(j).start()

    xcp = pltpu.make_async_copy(x_hbm, xbuf, x_sem)
    xcp.start()
    xcp.wait()
    sup[...] = lax.dot_general(
        xbuf[...].astype(jnp.bfloat16), w_ref[...].astype(jnp.bfloat16),
        (((1,), (0,)), ((), ())),
        preferred_element_type=jnp.float32).astype(jnp.bfloat16)

    for j in range(n):
        sz = blocks[j]
        adj_copy(j).wait()
        a = abuf[j % _NBUF, pl.ds(0, sz), :].astype(jnp.bfloat16)
        acc = lax.dot_general(
            a, sup[...], (((1,), (0,)), ((), ())),
            preferred_element_type=jnp.float32)
        if j >= 2:
            out_copy(j - 2).wait()
        obuf[j % 2, pl.ds(0, sz), :] = jnp.maximum(acc + b_ref[...], 0.0)
        out_copy(j).start()
        if j + _NBUF < n:
            adj_copy(j + _NBUF).start()

    for j in range(max(n - 2, 0), n):
        out_copy(j).wait()


@jax.jit
def kernel(x, adj, W, b):
    M, K = adj.shape
    D_in = x.shape[1]
    D_out = W.shape[1]

    BM = 272
    blocks = []
    rem = M
    while rem > 0:
        sz = min(BM, rem)
        blocks.append(sz)
        rem -= sz
    offs = [sum(blocks[:j]) for j in range(len(blocks))]

    out = pl.pallas_call(
        functools.partial(_persistent_body, blocks, offs),
        in_specs=[
            pl.BlockSpec((D_in, D_out), lambda: (0, 0)),
            pl.BlockSpec((1, D_out), lambda: (0, 0)),
            pl.BlockSpec(memory_space=pl.ANY),
            pl.BlockSpec(memory_space=pl.ANY),
        ],
        out_specs=pl.BlockSpec(memory_space=pl.ANY),
        out_shape=jax.ShapeDtypeStruct((M, D_out), jnp.float32),
        scratch_shapes=[
            pltpu.VMEM((K, D_in), jnp.float32),
            pltpu.VMEM((_NBUF, BM, K), jnp.float32),
            pltpu.VMEM((K, D_out), jnp.bfloat16),
            pltpu.VMEM((2, BM, D_out), jnp.float32),
            pltpu.SemaphoreType.DMA,
            pltpu.SemaphoreType.DMA((_NBUF,)),
            pltpu.SemaphoreType.DMA((2,)),
        ],
    )(W, b.reshape(1, D_out), x, adj)

    return out
